# Initial kernel scaffold; baseline (speedup 1.0000x reference)
import jax
import jax.numpy as jnp
from jax.experimental import pallas as pl


def kernel(x):
    raise NotImplementedError("write your pallas kernel here")


# speedup vs baseline: 5.7200x; 5.7200x over previous
import functools

import jax
import jax.numpy as jnp
import numpy as np
from jax import lax
from jax.experimental import pallas as pl
from jax.experimental.pallas import tpu as pltpu
from jax.experimental.pallas import tpu_sc as plsc

B = 32
N = 221184
K = 110592
L = 16
NC, NS = 2, 16
CHUNK = N // 8
NCHUNK = 8
VPC = CHUNK // L
NB1 = 2048
HISTW = NB1 * L

_SIGN = np.int32(-2147483648)


def _monotonic_key(v):
    b = plsc.bitcast(v, jnp.int32)
    m = lax.shift_right_arithmetic(b, 31)
    return lax.bitwise_xor(b, lax.bitwise_or(m, _SIGN))


_sc_mesh = plsc.VectorSubcoreMesh(
    core_axis_name="c", subcore_axis_name="s", num_cores=NC, num_subcores=NS
)


@functools.partial(
    pl.kernel,
    out_type=jax.ShapeDtypeStruct((B, L), jnp.float32),
    mesh=_sc_mesh,
    scratch_types=[
        pltpu.VMEM((CHUNK,), jnp.float32),
        pltpu.VMEM((CHUNK,), jnp.float32),
        pltpu.VMEM((HISTW,), jnp.int32),
        pltpu.VMEM((L,), jnp.float32),
        pltpu.SemaphoreType.DMA,
        pltpu.SemaphoreType.DMA,
    ],
    compiler_params=pltpu.CompilerParams(needs_layout_passes=False),
)
def _thresholds(x_hbm, out_hbm, buf0, buf1, hist, tbuf, sem0, sem1):
    wid = lax.axis_index("s") * NC + lax.axis_index("c")
    lanes = lax.iota(jnp.int32, L)
    ones = jnp.ones((L,), jnp.int32)

    def zero_hist(nwords):
        def zb(i, c):
            hist[pl.ds(i * L, L)] = jnp.zeros((L,), jnp.int32)
            return c
        lax.fori_loop(0, nwords // L, zb, jnp.int32(0))

    def count_and_find(level, prefix, kk):
        nbins = 1024 if level == 2 else NB1
        laneoff = lanes * nbins
        zero_hist(nbins * L)
        copies = [None] * NCHUNK
        copies[0] = pltpu.async_copy(x_hbm.at[wid, pl.ds(0, CHUNK)], buf0, sem0)
        for c in range(NCHUNK):
            buf = buf0 if c % 2 == 0 else buf1
            copies[c].wait()
            if c + 1 < NCHUNK:
                nbuf = buf1 if c % 2 == 0 else buf0
                nsem = sem1 if c % 2 == 0 else sem0
                copies[c + 1] = pltpu.async_copy(
                    x_hbm.at[wid, pl.ds((c + 1) * CHUNK, CHUNK)], nbuf, nsem
                )

            def body(i, carry):
                u = _monotonic_key(buf[pl.ds(i * L, L)])
                if level == 0:
                    binv = lax.shift_right_logical(u, 21)
                    msk = None
                elif level == 1:
                    binv = lax.bitwise_and(
                        lax.shift_right_logical(u, 10), np.int32(NB1 - 1))
                    msk = lax.shift_right_logical(u, 21) == prefix
                else:
                    binv = lax.bitwise_and(u, np.int32(1023))
                    msk = lax.shift_right_logical(u, 10) == prefix
                plsc.addupdate_scatter(hist, [laneoff + binv], ones, mask=msk)
                return carry

            lax.fori_loop(0, VPC, body, jnp.int32(0))

        nv = nbins // L

        def fb(j, carry):
            acc, cnt, mlow = carry
            jj = nv - 1 - j
            v = jnp.zeros((L,), jnp.int32)
            for l in range(L):
                v = v + hist[pl.ds(l * nbins + jj * L, L)]
            ps = plsc.cumsum(v)
            tot = jnp.max(ps)
            s = acc + (tot - ps) + v
            cond = s >= kk
            cnt = cnt + jnp.max(plsc.all_reduce_population_count(cond))
            mlow = jnp.maximum(mlow, jnp.max(jnp.where(cond, 0, s)))
            return acc + tot, cnt, mlow

        _, cnt, mlow = lax.fori_loop(
            0, nv, fb, (jnp.int32(0), jnp.int32(0), jnp.int32(0)))
        return cnt - 1, kk - mlow

    b1, k1 = count_and_find(0, jnp.int32(0), jnp.int32(K))
    b2, k2 = count_and_find(1, b1, k1)
    b3, _ = count_and_find(2, (b1 << 11) | b2, k2)

    u_th = (b1 << 21) | (b2 << 10) | b3
    t_bits = jnp.where(u_th < 0, u_th ^ _SIGN, ~u_th)
    tbuf[...] = plsc.bitcast(jnp.full((L,), t_bits, jnp.int32), jnp.float32)
    pltpu.sync_copy(tbuf, out_hbm.at[wid])


_RB = 432


def _mask_body(th_ref, x_ref, o_ref):
    t = th_ref[pl.program_id(0), 0]
    xv = x_ref[...]
    o_ref[...] = jnp.where(xv >= t, xv, jnp.float32(0.0))


@jax.jit
def kernel(x):
    xf = x.reshape(B, N)
    th = _thresholds(xf)
    x3 = xf.reshape(B, 1728, 128)
    out = pl.pallas_call(
        _mask_body,
        grid=(B, 1728 // _RB),
        in_specs=[
            pl.BlockSpec(memory_space=pltpu.SMEM),
            pl.BlockSpec((1, _RB, 128), lambda i, j: (i, j, 0)),
        ],
        out_specs=pl.BlockSpec((1, _RB, 128), lambda i, j: (i, j, 0)),
        out_shape=jax.ShapeDtypeStruct((B, 1728, 128), jnp.float32),
    )(th, x3)
    return out.reshape(x.shape)


# speedup vs baseline: 8.2592x; 1.4439x over previous
import functools

import jax
import jax.numpy as jnp
import numpy as np
from jax import lax
from jax.experimental import pallas as pl
from jax.experimental.pallas import tpu as pltpu
from jax.experimental.pallas import tpu_sc as plsc

B = 32
N = 221184
K = 110592
L = 16
NC, NS = 2, 16
CHUNK = N // 8
NCHUNK = 8
VPC = CHUNK // L
NB1 = 2048
HISTW = NB1 * L

_SIGN = np.int32(-2147483648)


def _monotonic_key(v):
    b = plsc.bitcast(v, jnp.int32)
    m = lax.shift_right_arithmetic(b, 31)
    return lax.bitwise_xor(b, lax.bitwise_or(m, _SIGN))


_sc_mesh = plsc.VectorSubcoreMesh(
    core_axis_name="c", subcore_axis_name="s", num_cores=NC, num_subcores=NS
)


@functools.partial(
    pl.kernel,
    out_type=jax.ShapeDtypeStruct((B, L), jnp.float32),
    mesh=_sc_mesh,
    scratch_types=[
        pltpu.VMEM((CHUNK,), jnp.float32),
        pltpu.VMEM((CHUNK,), jnp.float32),
        pltpu.VMEM((HISTW,), jnp.int32),
        pltpu.VMEM((L,), jnp.float32),
        pltpu.SemaphoreType.DMA,
        pltpu.SemaphoreType.DMA,
    ],
    compiler_params=pltpu.CompilerParams(needs_layout_passes=False),
)
def _thresholds(x_hbm, out_hbm, buf0, buf1, hist, tbuf, sem0, sem1):
    wid = lax.axis_index("s") * NC + lax.axis_index("c")
    lanes = lax.iota(jnp.int32, L)
    ones = jnp.ones((L,), jnp.int32)

    def zero_hist(nwords):
        @plsc.parallel_loop(0, nwords // L, unroll=8)
        def _zb(i):
            hist[pl.ds(i * L, L)] = jnp.zeros((L,), jnp.int32)

    def count_and_find(level, prefix, kk):
        nbins = 1024 if level == 2 else NB1
        laneoff = lanes * nbins
        zero_hist(nbins * L)
        copies = [None] * NCHUNK
        copies[0] = pltpu.async_copy(x_hbm.at[wid, pl.ds(0, CHUNK)], buf0, sem0)
        for c in range(NCHUNK):
            buf = buf0 if c % 2 == 0 else buf1
            copies[c].wait()
            if c + 1 < NCHUNK:
                nbuf = buf1 if c % 2 == 0 else buf0
                nsem = sem1 if c % 2 == 0 else sem0
                copies[c + 1] = pltpu.async_copy(
                    x_hbm.at[wid, pl.ds((c + 1) * CHUNK, CHUNK)], nbuf, nsem
                )

            @plsc.parallel_loop(0, VPC, unroll=8)
            def _body(i):
                u = _monotonic_key(buf[pl.ds(i * L, L)])
                if level == 0:
                    binv = lax.shift_right_logical(u, 21)
                    msk = None
                elif level == 1:
                    binv = lax.bitwise_and(
                        lax.shift_right_logical(u, 10), np.int32(NB1 - 1))
                    msk = lax.shift_right_logical(u, 21) == prefix
                else:
                    binv = lax.bitwise_and(u, np.int32(1023))
                    msk = lax.shift_right_logical(u, 10) == prefix
                plsc.addupdate_scatter(hist, [laneoff + binv], ones, mask=msk)

        nv = nbins // L

        @plsc.parallel_loop(
            0, nv, unroll=2,
            carry=(jnp.int32(0), jnp.int32(0), jnp.int32(0)))
        def fb(j, carry):
            acc, cnt, mlow = carry
            jj = nv - 1 - j
            v = jnp.zeros((L,), jnp.int32)
            for l in range(L):
                v = v + hist[pl.ds(l * nbins + jj * L, L)]
            ps = plsc.cumsum(v)
            tot = jnp.max(ps)
            s = acc + (tot - ps) + v
            cond = s >= kk
            cnt = cnt + jnp.max(plsc.all_reduce_population_count(cond))
            mlow = jnp.maximum(mlow, jnp.max(jnp.where(cond, 0, s)))
            return acc + tot, cnt, mlow

        _, cnt, mlow = fb
        return cnt - 1, kk - mlow

    b1, k1 = count_and_find(0, jnp.int32(0), jnp.int32(K))
    b2, k2 = count_and_find(1, b1, k1)
    b3, _ = count_and_find(2, (b1 << 11) | b2, k2)

    u_th = (b1 << 21) | (b2 << 10) | b3
    t_bits = jnp.where(u_th < 0, u_th ^ _SIGN, ~u_th)
    tbuf[...] = plsc.bitcast(jnp.full((L,), t_bits, jnp.int32), jnp.float32)
    pltpu.sync_copy(tbuf, out_hbm.at[wid])


_RB = 432


def _mask_body(th_ref, x_ref, o_ref):
    t = th_ref[pl.program_id(0), 0]
    xv = x_ref[...]
    o_ref[...] = jnp.where(xv >= t, xv, jnp.float32(0.0))


@jax.jit
def kernel(x):
    xf = x.reshape(B, N)
    th = _thresholds(xf)
    x3 = xf.reshape(B, 1728, 128)
    out = pl.pallas_call(
        _mask_body,
        grid=(B, 1728 // _RB),
        in_specs=[
            pl.BlockSpec(memory_space=pltpu.SMEM),
            pl.BlockSpec((1, _RB, 128), lambda i, j: (i, j, 0)),
        ],
        out_specs=pl.BlockSpec((1, _RB, 128), lambda i, j: (i, j, 0)),
        out_shape=jax.ShapeDtypeStruct((B, 1728, 128), jnp.float32),
    )(th, x3)
    return out.reshape(x.shape)


# speedup vs baseline: 34.3505x; 4.1591x over previous
import functools

import jax
import jax.numpy as jnp
import numpy as np
from jax import lax
from jax.experimental import pallas as pl
from jax.experimental.pallas import tpu as pltpu
from jax.experimental.pallas import tpu_sc as plsc

B = 32
N = 221184
K = 110592
L = 16
NC, NS = 2, 16
CHUNK = N // 8
NCHUNK = 8
VPC = CHUNK // L
NB1 = 2048
HISTW = NB1 * L

_SIGN = np.int32(-2147483648)


def _monotonic_key(v):
    b = plsc.bitcast(v, jnp.int32)
    m = lax.shift_right_arithmetic(b, 31)
    return lax.bitwise_xor(b, lax.bitwise_or(m, _SIGN))


_sc_mesh = plsc.VectorSubcoreMesh(
    core_axis_name="c", subcore_axis_name="s", num_cores=NC, num_subcores=NS
)


@functools.partial(
    pl.kernel,
    out_type=jax.ShapeDtypeStruct((B, L), jnp.float32),
    mesh=_sc_mesh,
    scratch_types=[
        pltpu.VMEM((CHUNK,), jnp.float32),
        pltpu.VMEM((CHUNK,), jnp.float32),
        pltpu.VMEM((HISTW,), jnp.int32),
        pltpu.VMEM((L,), jnp.float32),
        pltpu.SemaphoreType.DMA,
        pltpu.SemaphoreType.DMA,
    ],
    compiler_params=pltpu.CompilerParams(needs_layout_passes=False),
)
def _thresholds(x_hbm, out_hbm, buf0, buf1, hist, tbuf, sem0, sem1):
    wid = lax.axis_index("s") * NC + lax.axis_index("c")
    lanes = lax.iota(jnp.int32, L)
    ones = jnp.ones((L,), jnp.int32)

    def zero_hist(nwords):
        @plsc.parallel_loop(0, nwords // L, unroll=8)
        def _zb(i):
            hist[pl.ds(i * L, L)] = jnp.zeros((L,), jnp.int32)

    def count_and_find(level, prefix, kk):
        nbins = 1024 if level == 2 else NB1
        laneoff = lanes * nbins
        zero_hist(nbins * L)
        copies = [None] * NCHUNK
        copies[0] = pltpu.async_copy(x_hbm.at[wid, pl.ds(0, CHUNK)], buf0, sem0)
        for c in range(NCHUNK):
            buf = buf0 if c % 2 == 0 else buf1
            copies[c].wait()
            if c + 1 < NCHUNK:
                nbuf = buf1 if c % 2 == 0 else buf0
                nsem = sem1 if c % 2 == 0 else sem0
                copies[c + 1] = pltpu.async_copy(
                    x_hbm.at[wid, pl.ds((c + 1) * CHUNK, CHUNK)], nbuf, nsem
                )

            @plsc.parallel_loop(0, VPC, unroll=8)
            def _body(i):
                u = _monotonic_key(buf[pl.ds(i * L, L)])
                if level == 0:
                    binv = lax.shift_right_logical(u, 21)
                    msk = None
                elif level == 1:
                    binv = lax.bitwise_and(
                        lax.shift_right_logical(u, 10), np.int32(NB1 - 1))
                    msk = lax.shift_right_logical(u, 21) == prefix
                else:
                    binv = lax.bitwise_and(u, np.int32(1023))
                    msk = lax.shift_right_logical(u, 10) == prefix
                plsc.addupdate_scatter(hist, [laneoff + binv], ones, mask=msk)

        nv = nbins // L

        @plsc.parallel_loop(
            0, nv, unroll=2,
            carry=(jnp.int32(0), jnp.int32(0), jnp.int32(0)))
        def fb(j, carry):
            acc, cnt, mlow = carry
            jj = nv - 1 - j
            v = jnp.zeros((L,), jnp.int32)
            for l in range(L):
                v = v + hist[pl.ds(l * nbins + jj * L, L)]
            ps = plsc.cumsum(v)
            tot = jnp.max(ps)
            s = acc + (tot - ps) + v
            cond = s >= kk
            cnt = cnt + jnp.max(plsc.all_reduce_population_count(cond))
            mlow = jnp.maximum(mlow, jnp.max(jnp.where(cond, 0, s)))
            return acc + tot, cnt, mlow

        _, cnt, mlow = fb
        return cnt - 1, kk - mlow

    b1, k1 = count_and_find(0, jnp.int32(0), jnp.int32(K))
    b2, k2 = count_and_find(1, b1, k1)
    b3, _ = count_and_find(2, (b1 << 11) | b2, k2)

    u_th = (b1 << 21) | (b2 << 10) | b3
    t_bits = jnp.where(u_th < 0, u_th ^ _SIGN, ~u_th)
    tbuf[...] = plsc.bitcast(jnp.full((L,), t_bits, jnp.int32), jnp.float32)
    pltpu.sync_copy(tbuf, out_hbm.at[wid])


def _mask_body(th_ref, x_ref, o_ref):
    t = th_ref[pl.program_id(0), 0]
    xv = x_ref[...]
    o_ref[...] = jnp.where(xv >= t, xv, jnp.float32(0.0))


@jax.jit
def kernel(x):
    xv = (
        x.reshape(B, 3, 128, 24, 3, 8)
        .transpose(0, 3, 4, 1, 5, 2)
        .reshape(B, N)
    )
    th = _thresholds(xv)
    xm = x.transpose(0, 2, 3, 1).reshape(B, 576, 384)
    out3 = pl.pallas_call(
        _mask_body,
        grid=(B, 3),
        in_specs=[
            pl.BlockSpec(memory_space=pltpu.SMEM),
            pl.BlockSpec((1, 576, 128), lambda i, j: (i, 0, j)),
        ],
        out_specs=pl.BlockSpec((1, 576, 128), lambda i, j: (i, 0, j)),
        out_shape=jax.ShapeDtypeStruct((B, 576, 384), jnp.float32),
    )(th, xm)
    return out3.reshape(B, 24, 24, 384).transpose(0, 3, 1, 2)


# speedup vs baseline: 36.8507x; 1.0728x over previous
import functools

import jax
import jax.numpy as jnp
import numpy as np
from jax import lax
from jax.experimental import pallas as pl
from jax.experimental.pallas import tpu as pltpu
from jax.experimental.pallas import tpu_sc as plsc

B = 32
N = 221184
K = 110592
L = 16
NC, NS = 2, 16
CHUNK = N // 8
NCHUNK = 8
CROWS = CHUNK // 128
NB1 = 2048
HISTW = NB1 * L

_SIGN = np.int32(-2147483648)


def _monotonic_key(v):
    b = plsc.bitcast(v, jnp.int32)
    m = lax.shift_right_arithmetic(b, 31)
    return lax.bitwise_xor(b, lax.bitwise_or(m, _SIGN))


_sc_mesh = plsc.VectorSubcoreMesh(
    core_axis_name="c", subcore_axis_name="s", num_cores=NC, num_subcores=NS
)


@functools.partial(
    pl.kernel,
    out_type=jax.ShapeDtypeStruct((B, L), jnp.float32),
    mesh=_sc_mesh,
    scratch_types=[
        pltpu.VMEM((CROWS, 128), jnp.float32),
        pltpu.VMEM((CROWS, 128), jnp.float32),
        pltpu.VMEM((HISTW,), jnp.int32),
        pltpu.VMEM((L,), jnp.float32),
        pltpu.SemaphoreType.DMA,
        pltpu.SemaphoreType.DMA,
    ],
    compiler_params=pltpu.CompilerParams(needs_layout_passes=False),
)
def _thresholds(x_hbm, out_hbm, buf0, buf1, hist, tbuf, sem0, sem1):
    wid = lax.axis_index("s") * NC + lax.axis_index("c")
    lanes = lax.iota(jnp.int32, L)
    ones = jnp.ones((L,), jnp.int32)

    def zero_hist(nwords):
        @plsc.parallel_loop(0, nwords // L, unroll=8)
        def _zb(i):
            hist[pl.ds(i * L, L)] = jnp.zeros((L,), jnp.int32)

    def count_and_find(level, prefix, kk):
        nbins = 1024 if level == 2 else NB1
        laneoff = lanes * nbins
        zero_hist(nbins * L)
        copies = [None] * NCHUNK
        copies[0] = pltpu.async_copy(
            x_hbm.at[wid, pl.ds(0, CROWS), :], buf0, sem0)
        for c in range(NCHUNK):
            buf = buf0 if c % 2 == 0 else buf1
            copies[c].wait()
            if c + 1 < NCHUNK:
                nbuf = buf1 if c % 2 == 0 else buf0
                nsem = sem1 if c % 2 == 0 else sem0
                copies[c + 1] = pltpu.async_copy(
                    x_hbm.at[wid, pl.ds((c + 1) * CROWS, CROWS), :], nbuf, nsem
                )

            @plsc.parallel_loop(0, CROWS, unroll=2)
            def _body(i):
                for c8 in range(8):
                    u = _monotonic_key(buf[i, pl.ds(c8 * L, L)])
                    if level == 0:
                        binv = lax.shift_right_logical(u, 21)
                        msk = None
                    elif level == 1:
                        binv = lax.bitwise_and(
                            lax.shift_right_logical(u, 10), np.int32(NB1 - 1))
                        msk = lax.shift_right_logical(u, 21) == prefix
                    else:
                        binv = lax.bitwise_and(u, np.int32(1023))
                        msk = lax.shift_right_logical(u, 10) == prefix
                    plsc.addupdate_scatter(
                        hist, [laneoff + binv], ones, mask=msk)

        nv = nbins // L

        @plsc.parallel_loop(
            0, nv, unroll=2,
            carry=(jnp.int32(0), jnp.int32(0), jnp.int32(0)))
        def fb(j, carry):
            acc, cnt, mlow = carry
            jj = nv - 1 - j
            v = jnp.zeros((L,), jnp.int32)
            for l in range(L):
                v = v + hist[pl.ds(l * nbins + jj * L, L)]
            ps = plsc.cumsum(v)
            tot = jnp.max(ps)
            s = acc + (tot - ps) + v
            cond = s >= kk
            cnt = cnt + jnp.max(plsc.all_reduce_population_count(cond))
            mlow = jnp.maximum(mlow, jnp.max(jnp.where(cond, 0, s)))
            return acc + tot, cnt, mlow

        _, cnt, mlow = fb
        return cnt - 1, kk - mlow

    b1, k1 = count_and_find(0, jnp.int32(0), jnp.int32(K))
    b2, k2 = count_and_find(1, b1, k1)
    b3, _ = count_and_find(2, (b1 << 11) | b2, k2)

    u_th = (b1 << 21) | (b2 << 10) | b3
    t_bits = jnp.where(u_th < 0, u_th ^ _SIGN, ~u_th)
    tbuf[...] = plsc.bitcast(jnp.full((L,), t_bits, jnp.int32), jnp.float32)
    pltpu.sync_copy(tbuf, out_hbm.at[wid])


def _mask_body(th_ref, x_ref, o_ref):
    t = th_ref[pl.program_id(0), 0]
    xv = x_ref[...]
    o_ref[...] = jnp.where(xv >= t, xv, jnp.float32(0.0))


@jax.jit
def kernel(x):
    xv = (
        x.reshape(B, 3, 128, 24, 3, 8)
        .transpose(0, 3, 4, 1, 5, 2)
        .reshape(B, 1728, 128)
    )
    th = _thresholds(xv)
    xm = x.transpose(0, 2, 3, 1).reshape(B, 576, 384)
    out3 = pl.pallas_call(
        _mask_body,
        grid=(B, 3),
        in_specs=[
            pl.BlockSpec(memory_space=pltpu.SMEM),
            pl.BlockSpec((1, 576, 128), lambda i, j: (i, 0, j)),
        ],
        out_specs=pl.BlockSpec((1, 576, 128), lambda i, j: (i, 0, j)),
        out_shape=jax.ShapeDtypeStruct((B, 576, 384), jnp.float32),
    )(th, xm)
    return out3.reshape(B, 24, 24, 384).transpose(0, 3, 1, 2)


# speedup vs baseline: 48.1966x; 1.3079x over previous
import functools

import jax
import jax.numpy as jnp
import numpy as np
from jax import lax
from jax.experimental import pallas as pl
from jax.experimental.pallas import tpu as pltpu
from jax.experimental.pallas import tpu_sc as plsc

B = 32
N = 221184
K = 110592
L = 16
NC, NS = 2, 16
CHUNK = N // 8
NCHUNK = 8
CROWS = CHUNK // 128
NB1 = 2048
HISTW = NB1 * L

_SIGN = np.int32(-2147483648)


def _monotonic_key(v):
    b = plsc.bitcast(v, jnp.int32)
    m = lax.shift_right_arithmetic(b, 31)
    return lax.bitwise_xor(b, lax.bitwise_or(m, _SIGN))


_sc_mesh = plsc.VectorSubcoreMesh(
    core_axis_name="c", subcore_axis_name="s", num_cores=NC, num_subcores=NS
)


@functools.partial(
    pl.kernel,
    out_type=jax.ShapeDtypeStruct((B, 1728, 128), jnp.float32),
    mesh=_sc_mesh,
    scratch_types=[
        pltpu.VMEM((CROWS, 128), jnp.float32),
        pltpu.VMEM((CROWS, 128), jnp.float32),
        pltpu.VMEM((HISTW,), jnp.int32),
        pltpu.SemaphoreType.DMA,
        pltpu.SemaphoreType.DMA,
        pltpu.SemaphoreType.DMA,
        pltpu.SemaphoreType.DMA,
    ],
    compiler_params=pltpu.CompilerParams(needs_layout_passes=False),
)
def _crelu(x_hbm, out_hbm, buf0, buf1, hist, sem0, sem1, osem0, osem1):
    wid = lax.axis_index("s") * NC + lax.axis_index("c")
    lanes = lax.iota(jnp.int32, L)
    ones = jnp.ones((L,), jnp.int32)

    def zero_hist(nwords):
        @plsc.parallel_loop(0, nwords // L, unroll=8)
        def _zb(i):
            hist[pl.ds(i * L, L)] = jnp.zeros((L,), jnp.int32)

    def count_and_find(level, prefix, kk):
        nbins = 1024 if level == 2 else NB1
        laneoff = lanes * nbins
        zero_hist(nbins * L)
        copies = [None] * NCHUNK
        copies[0] = pltpu.async_copy(
            x_hbm.at[wid, pl.ds(0, CROWS), :], buf0, sem0)
        for c in range(NCHUNK):
            buf = buf0 if c % 2 == 0 else buf1
            copies[c].wait()
            if c + 1 < NCHUNK:
                nbuf = buf1 if c % 2 == 0 else buf0
                nsem = sem1 if c % 2 == 0 else sem0
                copies[c + 1] = pltpu.async_copy(
                    x_hbm.at[wid, pl.ds((c + 1) * CROWS, CROWS), :], nbuf, nsem
                )

            @plsc.parallel_loop(0, CROWS, unroll=2)
            def _body(i):
                for c8 in range(8):
                    u = _monotonic_key(buf[i, pl.ds(c8 * L, L)])
                    if level == 0:
                        binv = lax.shift_right_logical(u, 21)
                        msk = None
                    elif level == 1:
                        binv = lax.bitwise_and(
                            lax.shift_right_logical(u, 10), np.int32(NB1 - 1))
                        msk = lax.shift_right_logical(u, 21) == prefix
                    else:
                        binv = lax.bitwise_and(u, np.int32(1023))
                        msk = lax.shift_right_logical(u, 10) == prefix
                    plsc.addupdate_scatter(
                        hist, [laneoff + binv], ones, mask=msk)

        nv = nbins // L

        @plsc.parallel_loop(
            0, nv, unroll=2,
            carry=(jnp.int32(0), jnp.int32(0), jnp.int32(0)))
        def fb(j, carry):
            acc, cnt, mlow = carry
            jj = nv - 1 - j
            v = jnp.zeros((L,), jnp.int32)
            for l in range(L):
                v = v + hist[pl.ds(l * nbins + jj * L, L)]
            ps = plsc.cumsum(v)
            tot = jnp.max(ps)
            s = acc + (tot - ps) + v
            cond = s >= kk
            cnt = cnt + jnp.max(plsc.all_reduce_population_count(cond))
            mlow = jnp.maximum(mlow, jnp.max(jnp.where(cond, 0, s)))
            return acc + tot, cnt, mlow

        _, cnt, mlow = fb
        return cnt - 1, kk - mlow

    b1, k1 = count_and_find(0, jnp.int32(0), jnp.int32(K))
    b2, k2 = count_and_find(1, b1, k1)
    b3, _ = count_and_find(2, (b1 << 11) | b2, k2)

    u_th = (b1 << 21) | (b2 << 10) | b3
    t_bits = jnp.where(u_th < 0, u_th ^ _SIGN, ~u_th)
    tvec = plsc.bitcast(jnp.full((L,), t_bits, jnp.int32), jnp.float32)

    zero = jnp.zeros((L,), jnp.float32)
    copies = [None] * NCHUNK
    ocopies = [None] * NCHUNK
    copies[0] = pltpu.async_copy(x_hbm.at[wid, pl.ds(0, CROWS), :], buf0, sem0)
    for c in range(NCHUNK):
        buf = buf0 if c % 2 == 0 else buf1
        copies[c].wait()
        if c + 1 < NCHUNK:
            nbuf = buf1 if c % 2 == 0 else buf0
            nsem = sem1 if c % 2 == 0 else sem0
            if c + 1 >= 2:
                ocopies[c - 1].wait()
            copies[c + 1] = pltpu.async_copy(
                x_hbm.at[wid, pl.ds((c + 1) * CROWS, CROWS), :], nbuf, nsem)

        @plsc.parallel_loop(0, CROWS, unroll=2)
        def _mask(i):
            for c8 in range(8):
                v = buf[i, pl.ds(c8 * L, L)]
                buf[i, pl.ds(c8 * L, L)] = jnp.where(v >= tvec, v, zero)

        osem = osem0 if c % 2 == 0 else osem1
        ocopies[c] = pltpu.async_copy(
            buf, out_hbm.at[wid, pl.ds(c * CROWS, CROWS), :], osem)
    ocopies[NCHUNK - 2].wait()
    ocopies[NCHUNK - 1].wait()


@jax.jit
def kernel(x):
    xv = (
        x.reshape(B, 3, 128, 24, 3, 8)
        .transpose(0, 3, 4, 1, 5, 2)
        .reshape(B, 1728, 128)
    )
    outv = _crelu(xv)
    return (
        outv.reshape(B, 24, 3, 3, 8, 128)
        .transpose(0, 3, 5, 1, 2, 4)
        .reshape(B, 384, 24, 24)
    )


# speedup vs baseline: 50.2663x; 1.0429x over previous
import functools

import jax
import jax.numpy as jnp
import numpy as np
from jax import lax
from jax.experimental import pallas as pl
from jax.experimental.pallas import tpu as pltpu
from jax.experimental.pallas import tpu_sc as plsc

B = 32
N = 221184
K = 110592
L = 16
NC, NS = 2, 16
CHUNK = N // 8
NCHUNK = 8
CROWS = CHUNK // 128
NB1 = 2048
HISTW = NB1 * L

_SIGN = np.int32(-2147483648)


def _monotonic_key(v):
    b = plsc.bitcast(v, jnp.int32)
    m = lax.shift_right_arithmetic(b, 31)
    return lax.bitwise_xor(b, lax.bitwise_or(m, _SIGN))


_sc_mesh = plsc.VectorSubcoreMesh(
    core_axis_name="c", subcore_axis_name="s", num_cores=NC, num_subcores=NS
)


@functools.partial(
    pl.kernel,
    out_type=jax.ShapeDtypeStruct((B, 1728, 128), jnp.float32),
    mesh=_sc_mesh,
    scratch_types=[
        pltpu.VMEM((CROWS, 128), jnp.float32),
        pltpu.VMEM((CROWS, 128), jnp.float32),
        pltpu.VMEM((HISTW,), jnp.int32),
        pltpu.SemaphoreType.DMA,
        pltpu.SemaphoreType.DMA,
        pltpu.SemaphoreType.DMA,
        pltpu.SemaphoreType.DMA,
    ],
    compiler_params=pltpu.CompilerParams(needs_layout_passes=False),
)
def _crelu(x_hbm, out_hbm, buf0, buf1, hist, sem0, sem1, osem0, osem1):
    wid = lax.axis_index("s") * NC + lax.axis_index("c")
    lanes = lax.iota(jnp.int32, L)
    ones = jnp.ones((L,), jnp.int32)

    def zero_hist(nwords):
        @plsc.parallel_loop(0, nwords // L, unroll=8)
        def _zb(i):
            hist[pl.ds(i * L, L)] = jnp.zeros((L,), jnp.int32)

    def count_and_find(level, prefix, kk, pending):
        nbins = 1024 if level == 2 else NB1
        laneoff = lanes * nbins
        zero_hist(nbins * L)
        copies = [None] * NCHUNK
        copies[0] = pending
        for c in range(NCHUNK):
            buf = buf0 if c % 2 == 0 else buf1
            copies[c].wait()
            if c + 1 < NCHUNK:
                nbuf = buf1 if c % 2 == 0 else buf0
                nsem = sem1 if c % 2 == 0 else sem0
                copies[c + 1] = pltpu.async_copy(
                    x_hbm.at[wid, pl.ds((c + 1) * CROWS, CROWS), :], nbuf, nsem
                )

            @plsc.parallel_loop(0, CROWS, unroll=2)
            def _body(i):
                for c8 in range(8):
                    u = _monotonic_key(buf[i, pl.ds(c8 * L, L)])
                    if level == 0:
                        binv = lax.shift_right_logical(u, 21)
                        msk = None
                    elif level == 1:
                        binv = lax.bitwise_and(
                            lax.shift_right_logical(u, 10), np.int32(NB1 - 1))
                        msk = lax.shift_right_logical(u, 21) == prefix
                    else:
                        binv = lax.bitwise_and(u, np.int32(1023))
                        msk = lax.shift_right_logical(u, 10) == prefix
                    plsc.addupdate_scatter(
                        hist, [laneoff + binv], ones, mask=msk)

        nxt = pltpu.async_copy(x_hbm.at[wid, pl.ds(0, CROWS), :], buf0, sem0)

        nv = nbins // L

        @plsc.parallel_loop(
            0, nv, unroll=2,
            carry=(jnp.int32(0), jnp.int32(0), jnp.int32(0)))
        def fb(j, carry):
            acc, cnt, mlow = carry
            jj = nv - 1 - j
            v = jnp.zeros((L,), jnp.int32)
            for l in range(L):
                v = v + hist[pl.ds(l * nbins + jj * L, L)]
            ps = plsc.cumsum(v)
            tot = jnp.max(ps)
            s = acc + (tot - ps) + v
            cond = s >= kk
            cnt = cnt + jnp.max(plsc.all_reduce_population_count(cond))
            mlow = jnp.maximum(mlow, jnp.max(jnp.where(cond, 0, s)))
            return acc + tot, cnt, mlow

        _, cnt, mlow = fb
        return cnt - 1, kk - mlow, nxt

    first = pltpu.async_copy(x_hbm.at[wid, pl.ds(0, CROWS), :], buf0, sem0)
    b1, k1, p1 = count_and_find(0, jnp.int32(0), jnp.int32(K), first)
    b2, k2, p2 = count_and_find(1, b1, k1, p1)
    b3, _, p3 = count_and_find(2, (b1 << 11) | b2, k2, p2)

    u_th = (b1 << 21) | (b2 << 10) | b3
    t_bits = jnp.where(u_th < 0, u_th ^ _SIGN, ~u_th)
    tvec = plsc.bitcast(jnp.full((L,), t_bits, jnp.int32), jnp.float32)

    zero = jnp.zeros((L,), jnp.float32)
    copies = [None] * NCHUNK
    ocopies = [None] * NCHUNK
    copies[0] = p3
    for c in range(NCHUNK):
        buf = buf0 if c % 2 == 0 else buf1
        copies[c].wait()
        if c + 1 < NCHUNK:
            nbuf = buf1 if c % 2 == 0 else buf0
            nsem = sem1 if c % 2 == 0 else sem0
            if c + 1 >= 2:
                ocopies[c - 1].wait()
            copies[c + 1] = pltpu.async_copy(
                x_hbm.at[wid, pl.ds((c + 1) * CROWS, CROWS), :], nbuf, nsem)

        @plsc.parallel_loop(0, CROWS, unroll=2)
        def _mask(i):
            for c8 in range(8):
                v = buf[i, pl.ds(c8 * L, L)]
                buf[i, pl.ds(c8 * L, L)] = jnp.where(v >= tvec, v, zero)

        osem = osem0 if c % 2 == 0 else osem1
        ocopies[c] = pltpu.async_copy(
            buf, out_hbm.at[wid, pl.ds(c * CROWS, CROWS), :], osem)
    ocopies[NCHUNK - 2].wait()
    ocopies[NCHUNK - 1].wait()


@jax.jit
def kernel(x):
    xv = (
        x.reshape(B, 3, 128, 24, 3, 8)
        .transpose(0, 3, 4, 1, 5, 2)
        .reshape(B, 1728, 128)
    )
    outv = _crelu(xv)
    return (
        outv.reshape(B, 24, 3, 3, 8, 128)
        .transpose(0, 3, 5, 1, 2, 4)
        .reshape(B, 384, 24, 24)
    )
